# baseline (speedup 1.0000x reference)
import functools

import jax
import jax.numpy as jnp
import numpy as np
from jax import lax
from jax.experimental import pallas as pl
from jax.experimental.pallas import tpu as pltpu
from jax.experimental.pallas import tpu_sc as plsc

T = 8
M = 1.0
LAM = 0.01
BLK = 512
BLK2 = 2048
SC_CHUNK = 64


def _tc_a(v_ref, vh_ref, f_ref, neg_ref, h_ref, stats_ref, ortho_ref):
    pid = pl.program_id(0)

    @pl.when(pid == 0)
    def _init():
        f = f_ref[...]
        gram = lax.dot_general(f, f, (((1,), (1,)), ((), ())),
                               preferred_element_type=jnp.float32)
        k = gram.shape[0]
        rows = lax.broadcasted_iota(jnp.int32, gram.shape, 0)
        cols = lax.broadcasted_iota(jnp.int32, gram.shape, 1)
        tr = jnp.sum(jnp.where(rows == cols, gram, 0.0))
        val = jnp.sum(gram * gram) - 2.0 * tr + float(k)
        ortho_ref[...] = jnp.broadcast_to(val, (1, 1))

    vhat = vh_ref[...]
    v = v_ref[...]
    vh2 = jnp.sum(vhat * vhat, axis=1)
    td = jnp.sqrt(jnp.sum((vhat - v) ** 2, axis=1) + 1e-8)

    ones_row = jnp.ones((8, v.shape[1]), jnp.float32)
    neg = neg_ref[...]
    nn2 = lax.dot_general(ones_row, neg * neg, (((1,), (1,)), ((), ())),
                          preferred_element_type=jnp.float32)[0:1, :]
    ndots = lax.dot_general(vhat, neg, (((1,), (1,)), ((), ())),
                            preferred_element_type=jnp.float32)
    nd = jnp.sqrt(jnp.maximum(vh2[:, None] - 2.0 * ndots + nn2, 0.0) + 1e-8)
    c = jnp.mean(jnp.maximum(1.0 + td[:, None] - nd, 0.0), axis=1)

    f = f_ref[...]
    fn2 = lax.dot_general(ones_row, f * f, (((1,), (1,)), ((), ())),
                          preferred_element_type=jnp.float32)[0:1, :]
    dots = lax.dot_general(vhat, f, (((1,), (1,)), ((), ())),
                           preferred_element_type=jnp.float32)
    h_ref[...] = fn2 - 2.0 * dots

    zcol = jnp.zeros_like(td)
    stats_ref[...] = jnp.stack(
        [td, vh2, c, zcol, zcol, zcol, zcol, zcol], axis=1)


def _sc_topk(g_hbm, h_hbm, gt_hbm, ht_hbm, g_v, h_v, gt_v, ht_v):
    info = plsc.get_sparse_core_info()
    nc = info.num_cores
    nw = nc * info.num_subcores
    wid = lax.axis_index("s") * nc + lax.axis_index("c")
    rows_per_w = g_hbm.shape[0] // nw
    base = wid * rows_per_w
    kk = g_hbm.shape[1]
    nvec = kk // 16
    lane = lax.iota(jnp.int32, 16)

    def do_row(r, _):
        sorted_vecs = []
        for j in range(nvec):
            gv = g_v[r, pl.ds(j * 16, 16)]
            kv = (plsc.bitcast(gv, jnp.int32) & np.int32(~511)) \
                | (lane + np.int32(j * 16))
            sorted_vecs.append(jnp.sort(kv))
        while len(sorted_vecs) > 1:
            nxt = []
            for a, b in zip(sorted_vecs[0::2], sorted_vecs[1::2]):
                low = jnp.minimum(a, jnp.flip(b, 0))
                nxt.append(jnp.sort(low))
            sorted_vecs = nxt
        best = sorted_vecs[0]
        kidx = best & np.int32(511)
        gval = plsc.bitcast(best & np.int32(~511), jnp.float32)
        rvec = jnp.broadcast_to(r, (16,)).astype(jnp.int32)
        hval = plsc.load_gather(h_v, [rvec, kidx])
        off = pl.multiple_of(r * 16, 16)
        gt_v[pl.ds(off, 16)] = gval
        ht_v[pl.ds(off, 16)] = hval
        return 0

    nchunk = rows_per_w // SC_CHUNK
    for ci in range(nchunk):
        rowbase = base + ci * SC_CHUNK
        pltpu.sync_copy(g_hbm.at[pl.ds(rowbase, SC_CHUNK)], g_v)
        pltpu.sync_copy(h_hbm.at[pl.ds(rowbase, SC_CHUNK)], h_v)
        lax.fori_loop(0, SC_CHUNK, do_row, 0)
        pltpu.sync_copy(gt_v, gt_hbm.at[pl.ds(rowbase * 16, SC_CHUNK * 16)])
        pltpu.sync_copy(ht_v, ht_hbm.at[pl.ds(rowbase * 16, SC_CHUNK * 16)])


def _tc_b(stats_ref, gt_ref, ht_ref, mask_ref, ortho_ref, out_ref, acc):
    pid = pl.program_id(0)
    nblk = pl.num_programs(0)

    @pl.when(pid == 0)
    def _init():
        acc[0] = 0.0
        acc[1] = 0.0

    stats = stats_ref[...]
    td = stats[:, 0:1]
    vh2 = stats[:, 1:2]
    c = stats[:, 2]
    gt = gt_ref[...]
    ht = ht_ref[...]
    tcol = lax.broadcasted_iota(jnp.int32, gt.shape, 1)
    valid = tcol < T
    gtm = jnp.where(valid, gt, 0.0)
    s = jnp.sum(gtm, axis=1, keepdims=True)
    gn = gtm / (s + 1e-10)
    mt = M * (1.0 - gn) ** 2
    dist = jnp.sqrt(jnp.maximum(vh2 + ht, 0.0) + 1e-8)
    terms = jnp.where(valid, jnp.maximum(mt + td - dist, 0.0), 0.0)
    jt = jnp.sum(terms, axis=1) / float(T)

    mask = mask_ref[0, 0, :]
    acc[0] += jnp.sum(mask * (c + jt))
    acc[1] += jnp.sum(mask)

    @pl.when(pid == nblk - 1)
    def _fin():
        val = acc[0] / jnp.maximum(acc[1], 1.0) + LAM * ortho_ref[0, 0]
        out_ref[...] = jnp.broadcast_to(val, (1, 1))


@functools.partial(jax.jit, static_argnames=())
def kernel(v, vhat, d, g, F, negatives, mask):
    del d
    B, D = v.shape
    K = F.shape[0]
    N = negatives.shape[0]
    nblk = B // BLK

    h, stats, ortho = pl.pallas_call(
        _tc_a,
        grid=(nblk,),
        in_specs=[
            pl.BlockSpec((BLK, D), lambda i: (i, 0)),
            pl.BlockSpec((BLK, D), lambda i: (i, 0)),
            pl.BlockSpec((K, D), lambda i: (0, 0)),
            pl.BlockSpec((N, D), lambda i: (0, 0)),
        ],
        out_specs=[
            pl.BlockSpec((BLK, K), lambda i: (i, 0)),
            pl.BlockSpec((BLK, 8), lambda i: (i, 0)),
            pl.BlockSpec((1, 1), lambda i: (0, 0)),
        ],
        out_shape=[
            jax.ShapeDtypeStruct((B, K), jnp.float32),
            jax.ShapeDtypeStruct((B, 8), jnp.float32),
            jax.ShapeDtypeStruct((1, 1), jnp.float32),
        ],
    )(v, vhat, F, negatives)

    mesh = plsc.VectorSubcoreMesh(core_axis_name="c", subcore_axis_name="s")
    gt_flat, ht_flat = pl.kernel(
        _sc_topk,
        mesh=mesh,
        compiler_params=pltpu.CompilerParams(needs_layout_passes=False),
        out_type=[
            jax.ShapeDtypeStruct((B * 16,), jnp.float32),
            jax.ShapeDtypeStruct((B * 16,), jnp.float32),
        ],
        scratch_types=[
            pltpu.VMEM((SC_CHUNK, K), jnp.float32),
            pltpu.VMEM((SC_CHUNK, K), jnp.float32),
            pltpu.VMEM((SC_CHUNK * 16,), jnp.float32),
            pltpu.VMEM((SC_CHUNK * 16,), jnp.float32),
        ],
    )(g, h)
    gt16 = gt_flat.reshape(B, 16)
    ht16 = ht_flat.reshape(B, 16)

    nblk2 = B // BLK2
    maskf = mask.astype(jnp.float32).reshape(nblk2, 1, BLK2)
    out = pl.pallas_call(
        _tc_b,
        grid=(nblk2,),
        in_specs=[
            pl.BlockSpec((BLK2, 8), lambda i: (i, 0)),
            pl.BlockSpec((BLK2, 16), lambda i: (i, 0)),
            pl.BlockSpec((BLK2, 16), lambda i: (i, 0)),
            pl.BlockSpec((1, 1, BLK2), lambda i: (i, 0, 0)),
            pl.BlockSpec((1, 1), lambda i: (0, 0)),
        ],
        out_specs=pl.BlockSpec((1, 1), lambda i: (0, 0)),
        out_shape=jax.ShapeDtypeStruct((1, 1), jnp.float32),
        scratch_shapes=[pltpu.SMEM((2,), jnp.float32)],
    )(stats, gt16, ht16, maskf, ortho)
    return out.reshape(())


# speedup vs baseline: 1.1375x; 1.1375x over previous
import functools

import jax
import jax.numpy as jnp
import numpy as np
from jax import lax
from jax.experimental import pallas as pl
from jax.experimental.pallas import tpu as pltpu
from jax.experimental.pallas import tpu_sc as plsc

T = 8
M = 1.0
LAM = 0.01
BLK = 512
BLK2 = 2048
SC_CHUNK = 64


def _tc_a(v_ref, vh_ref, f_ref, neg_ref, h_ref, stats_ref, ortho_ref):
    pid = pl.program_id(0)

    @pl.when(pid == 0)
    def _init():
        f = f_ref[...]
        gram = lax.dot_general(f, f, (((1,), (1,)), ((), ())),
                               preferred_element_type=jnp.float32)
        k = gram.shape[0]
        rows = lax.broadcasted_iota(jnp.int32, gram.shape, 0)
        cols = lax.broadcasted_iota(jnp.int32, gram.shape, 1)
        tr = jnp.sum(jnp.where(rows == cols, gram, 0.0))
        val = jnp.sum(gram * gram) - 2.0 * tr + float(k)
        ortho_ref[...] = jnp.broadcast_to(val, (1, 1))

    vhat = vh_ref[...]
    v = v_ref[...]
    vh2 = jnp.sum(vhat * vhat, axis=1)
    td = jnp.sqrt(jnp.sum((vhat - v) ** 2, axis=1) + 1e-8)

    ones_row = jnp.ones((8, v.shape[1]), jnp.float32)
    neg = neg_ref[...]
    nn2 = lax.dot_general(ones_row, neg * neg, (((1,), (1,)), ((), ())),
                          preferred_element_type=jnp.float32)[0:1, :]
    ndots = lax.dot_general(vhat, neg, (((1,), (1,)), ((), ())),
                            preferred_element_type=jnp.float32)
    nd = jnp.sqrt(jnp.maximum(vh2[:, None] - 2.0 * ndots + nn2, 0.0) + 1e-8)
    c = jnp.mean(jnp.maximum(1.0 + td[:, None] - nd, 0.0), axis=1)

    f = f_ref[...]
    fn2 = lax.dot_general(ones_row, f * f, (((1,), (1,)), ((), ())),
                          preferred_element_type=jnp.float32)[0:1, :]
    dots = lax.dot_general(vhat, f, (((1,), (1,)), ((), ())),
                           preferred_element_type=jnp.float32)
    h_ref[...] = fn2 - 2.0 * dots

    zcol = jnp.zeros_like(td)
    stats_ref[...] = jnp.stack(
        [td, vh2, c, zcol, zcol, zcol, zcol, zcol], axis=1)


def _sc_topk(g_hbm, gt_hbm, idx_hbm, g_v, gt_v, idx_v):
    info = plsc.get_sparse_core_info()
    nc = info.num_cores
    nw = nc * info.num_subcores
    wid = lax.axis_index("s") * nc + lax.axis_index("c")
    rows_per_w = g_hbm.shape[0] // nw
    base = wid * rows_per_w
    kk = g_hbm.shape[1]
    nvec = kk // 16
    lane = lax.iota(jnp.int32, 16)

    def make_row_body(rowbase):
        del rowbase

        def do_row(r, _):
            sorted_vecs = []
            for j in range(nvec):
                gv = g_v[r, pl.ds(j * 16, 16)]
                kv = (plsc.bitcast(gv, jnp.int32) & np.int32(~511)) \
                    | (lane + np.int32(j * 16))
                sorted_vecs.append(jnp.sort(kv))
            while len(sorted_vecs) > 1:
                nxt = []
                for a, b in zip(sorted_vecs[0::2], sorted_vecs[1::2]):
                    low = jnp.minimum(a, jnp.flip(b, 0))
                    nxt.append(jnp.sort(low))
                sorted_vecs = nxt
            best = sorted_vecs[0]
            gval = plsc.bitcast(best & np.int32(~511), jnp.float32)
            off = pl.multiple_of(r * 16, 16)
            gt_v[pl.ds(off, 16)] = gval
            idx_v[pl.ds(off, 16)] = best & np.int32(511)
            return 0
        return do_row

    nchunk = rows_per_w // SC_CHUNK
    for ci in range(nchunk):
        rowbase = base + ci * SC_CHUNK
        pltpu.sync_copy(g_hbm.at[pl.ds(rowbase, SC_CHUNK)], g_v)
        lax.fori_loop(0, SC_CHUNK, make_row_body(rowbase), 0)
        pltpu.sync_copy(gt_v, gt_hbm.at[pl.ds(rowbase * 16, SC_CHUNK * 16)])
        pltpu.sync_copy(idx_v, idx_hbm.at[pl.ds(rowbase * 16, SC_CHUNK * 16)])


def _sc_gather(h_hbm, idx_hbm, ht_hbm, h_v, idx_v, ht_v):
    info = plsc.get_sparse_core_info()
    nc = info.num_cores
    nw = nc * info.num_subcores
    wid = lax.axis_index("s") * nc + lax.axis_index("c")
    nrows = h_hbm.shape[0]
    kk = h_hbm.shape[1]
    rows_per_w = nrows // nw
    base = wid * rows_per_w

    def do_row(r, _):
        off = pl.multiple_of(r * 16, 16)
        kidx = idx_v[pl.ds(off, 16)]
        rvec = jnp.broadcast_to(r, (16,)).astype(jnp.int32)
        ht_v[pl.ds(off, 16)] = plsc.load_gather(h_v, [rvec, kidx])
        return 0

    nchunk = rows_per_w // SC_CHUNK
    for ci in range(nchunk):
        rowbase = base + ci * SC_CHUNK
        pltpu.sync_copy(h_hbm.at[pl.ds(rowbase, SC_CHUNK)], h_v)
        pltpu.sync_copy(idx_hbm.at[pl.ds(rowbase * 16, SC_CHUNK * 16)], idx_v)
        lax.fori_loop(0, SC_CHUNK, do_row, 0)
        pltpu.sync_copy(ht_v, ht_hbm.at[pl.ds(rowbase * 16, SC_CHUNK * 16)])


def _tc_b(stats_ref, gt_ref, ht_ref, mask_ref, ortho_ref, out_ref, acc):
    pid = pl.program_id(0)
    nblk = pl.num_programs(0)

    @pl.when(pid == 0)
    def _init():
        acc[0] = 0.0
        acc[1] = 0.0

    stats = stats_ref[...]
    td = stats[:, 0:1]
    vh2 = stats[:, 1:2]
    c = stats[:, 2]
    gt = gt_ref[...]
    ht = ht_ref[...]
    tcol = lax.broadcasted_iota(jnp.int32, gt.shape, 1)
    valid = tcol < T
    gtm = jnp.where(valid, gt, 0.0)
    s = jnp.sum(gtm, axis=1, keepdims=True)
    gn = gtm / (s + 1e-10)
    mt = M * (1.0 - gn) ** 2
    dist = jnp.sqrt(jnp.maximum(vh2 + ht, 0.0) + 1e-8)
    terms = jnp.where(valid, jnp.maximum(mt + td - dist, 0.0), 0.0)
    jt = jnp.sum(terms, axis=1) / float(T)

    mask = mask_ref[0, 0, :]
    acc[0] += jnp.sum(mask * (c + jt))
    acc[1] += jnp.sum(mask)

    @pl.when(pid == nblk - 1)
    def _fin():
        val = acc[0] / jnp.maximum(acc[1], 1.0) + LAM * ortho_ref[0, 0]
        out_ref[...] = jnp.broadcast_to(val, (1, 1))


@functools.partial(jax.jit, static_argnames=())
def kernel(v, vhat, d, g, F, negatives, mask):
    del d
    B, D = v.shape
    K = F.shape[0]
    N = negatives.shape[0]
    nblk = B // BLK

    h, stats, ortho = pl.pallas_call(
        _tc_a,
        grid=(nblk,),
        in_specs=[
            pl.BlockSpec((BLK, D), lambda i: (i, 0)),
            pl.BlockSpec((BLK, D), lambda i: (i, 0)),
            pl.BlockSpec((K, D), lambda i: (0, 0)),
            pl.BlockSpec((N, D), lambda i: (0, 0)),
        ],
        out_specs=[
            pl.BlockSpec((BLK, K), lambda i: (i, 0)),
            pl.BlockSpec((BLK, 8), lambda i: (i, 0)),
            pl.BlockSpec((1, 1), lambda i: (0, 0)),
        ],
        out_shape=[
            jax.ShapeDtypeStruct((B, K), jnp.float32),
            jax.ShapeDtypeStruct((B, 8), jnp.float32),
            jax.ShapeDtypeStruct((1, 1), jnp.float32),
        ],
    )(v, vhat, F, negatives)

    mesh = plsc.VectorSubcoreMesh(core_axis_name="c", subcore_axis_name="s")
    gt_flat, idx_flat = pl.kernel(
        _sc_topk,
        mesh=mesh,
        compiler_params=pltpu.CompilerParams(needs_layout_passes=False),
        out_type=[
            jax.ShapeDtypeStruct((B * 16,), jnp.float32),
            jax.ShapeDtypeStruct((B * 16,), jnp.int32),
        ],
        scratch_types=[
            pltpu.VMEM((SC_CHUNK, K), jnp.float32),
            pltpu.VMEM((SC_CHUNK * 16,), jnp.float32),
            pltpu.VMEM((SC_CHUNK * 16,), jnp.int32),
        ],
    )(g)
    ht_flat = pl.kernel(
        _sc_gather,
        mesh=mesh,
        compiler_params=pltpu.CompilerParams(needs_layout_passes=False),
        out_type=jax.ShapeDtypeStruct((B * 16,), jnp.float32),
        scratch_types=[
            pltpu.VMEM((SC_CHUNK, K), jnp.float32),
            pltpu.VMEM((SC_CHUNK * 16,), jnp.int32),
            pltpu.VMEM((SC_CHUNK * 16,), jnp.float32),
        ],
    )(h, idx_flat)
    gt16 = gt_flat.reshape(B, 16)
    ht16 = ht_flat.reshape(B, 16)

    nblk2 = B // BLK2
    maskf = mask.astype(jnp.float32).reshape(nblk2, 1, BLK2)
    out = pl.pallas_call(
        _tc_b,
        grid=(nblk2,),
        in_specs=[
            pl.BlockSpec((BLK2, 8), lambda i: (i, 0)),
            pl.BlockSpec((BLK2, 16), lambda i: (i, 0)),
            pl.BlockSpec((BLK2, 16), lambda i: (i, 0)),
            pl.BlockSpec((1, 1, BLK2), lambda i: (i, 0, 0)),
            pl.BlockSpec((1, 1), lambda i: (0, 0)),
        ],
        out_specs=pl.BlockSpec((1, 1), lambda i: (0, 0)),
        out_shape=jax.ShapeDtypeStruct((1, 1), jnp.float32),
        scratch_shapes=[pltpu.SMEM((2,), jnp.float32)],
    )(stats, gt16, ht16, maskf, ortho)
    return out.reshape(())


# speedup vs baseline: 1.2332x; 1.0842x over previous
import functools

import jax
import jax.numpy as jnp
import numpy as np
from jax import lax
from jax.experimental import pallas as pl
from jax.experimental.pallas import tpu as pltpu
from jax.experimental.pallas import tpu_sc as plsc

T = 8
M = 1.0
LAM = 0.01
BLK = 512
SC_CHUNK = 64


def _tc_a(v_ref, vh_ref, f_ref, neg_ref, mask_ref,
          h_ref, td_ref, vh2_ref, ortho_ref, csum_ref, msum_ref, acc):
    pid = pl.program_id(0)
    nblk = pl.num_programs(0)

    @pl.when(pid == 0)
    def _init():
        f = f_ref[...]
        gram = lax.dot_general(f, f, (((1,), (1,)), ((), ())),
                               preferred_element_type=jnp.float32)
        k = gram.shape[0]
        rows = lax.broadcasted_iota(jnp.int32, gram.shape, 0)
        cols = lax.broadcasted_iota(jnp.int32, gram.shape, 1)
        tr = jnp.sum(jnp.where(rows == cols, gram, 0.0))
        acc[0] = jnp.sum(gram * gram) - 2.0 * tr + float(k)
        acc[1] = 0.0
        acc[2] = 0.0

    vhat = vh_ref[...]
    v = v_ref[...]
    vh2 = jnp.sum(vhat * vhat, axis=1)
    td = jnp.sqrt(jnp.sum((vhat - v) ** 2, axis=1) + 1e-8)

    ones_row = jnp.ones((8, v.shape[1]), jnp.float32)
    neg = neg_ref[...]
    nn2 = lax.dot_general(ones_row, neg * neg, (((1,), (1,)), ((), ())),
                          preferred_element_type=jnp.float32)[0:1, :]
    ndots = lax.dot_general(vhat, neg, (((1,), (1,)), ((), ())),
                            preferred_element_type=jnp.float32)
    nd = jnp.sqrt(jnp.maximum(vh2[:, None] - 2.0 * ndots + nn2, 0.0) + 1e-8)
    c = jnp.mean(jnp.maximum(1.0 + td[:, None] - nd, 0.0), axis=1)

    f = f_ref[...]
    fn2 = lax.dot_general(ones_row, f * f, (((1,), (1,)), ((), ())),
                          preferred_element_type=jnp.float32)[0:1, :]
    dots = lax.dot_general(vhat, f, (((1,), (1,)), ((), ())),
                           preferred_element_type=jnp.float32)
    h_ref[...] = fn2 - 2.0 * dots
    td_ref[...] = td
    vh2_ref[...] = vh2

    mask = mask_ref[0, 0, :]
    acc[1] += jnp.sum(mask * c)
    acc[2] += jnp.sum(mask)

    @pl.when(pid == nblk - 1)
    def _fin():
        ortho_ref[...] = jnp.broadcast_to(acc[0], (1, 1))
        csum_ref[...] = jnp.broadcast_to(acc[1], (1, 1))
        msum_ref[...] = jnp.broadcast_to(acc[2], (1, 1))


def _sc_topk(g_hbm, gt_hbm, idx_hbm, g_v, gt_v, idx_v):
    info = plsc.get_sparse_core_info()
    nc = info.num_cores
    nw = nc * info.num_subcores
    wid = lax.axis_index("s") * nc + lax.axis_index("c")
    rows_per_w = g_hbm.shape[0] // nw
    base = wid * rows_per_w
    kk = g_hbm.shape[1]
    nvec = kk // 16
    lane = lax.iota(jnp.int32, 16)

    def do_row(r, carry):
        sorted_vecs = []
        for j in range(nvec):
            gv = g_v[r, pl.ds(j * 16, 16)]
            kv = (plsc.bitcast(gv, jnp.int32) & np.int32(~511)) \
                | (lane + np.int32(j * 16))
            sorted_vecs.append(jnp.sort(kv))
        while len(sorted_vecs) > 1:
            nxt = []
            for a, b in zip(sorted_vecs[0::2], sorted_vecs[1::2]):
                low = jnp.minimum(a, jnp.flip(b, 0))
                nxt.append(jnp.sort(low))
            sorted_vecs = nxt
        best = sorted_vecs[0]
        gval = plsc.bitcast(best & np.int32(~511), jnp.float32)
        off = pl.multiple_of(r * 16, 16)
        gt_v[pl.ds(off, 16)] = gval
        idx_v[pl.ds(off, 16)] = best & np.int32(511)
        return carry

    nchunk = rows_per_w // SC_CHUNK
    for ci in range(nchunk):
        rowbase = base + ci * SC_CHUNK
        pltpu.sync_copy(g_hbm.at[pl.ds(rowbase, SC_CHUNK)], g_v)
        lax.fori_loop(0, SC_CHUNK, do_row, 0)
        pltpu.sync_copy(gt_v, gt_hbm.at[pl.ds(rowbase * 16, SC_CHUNK * 16)])
        pltpu.sync_copy(idx_v, idx_hbm.at[pl.ds(rowbase * 16, SC_CHUNK * 16)])


def _sc_jt(h_hbm, gt_hbm, idx_hbm, td_hbm, vh2_hbm, mk_hbm, part_hbm,
           h_v, gt_v, idx_v, td_v, vh2_v, mk_v, out_v):
    info = plsc.get_sparse_core_info()
    nc = info.num_cores
    nw = nc * info.num_subcores
    wid = lax.axis_index("s") * nc + lax.axis_index("c")
    rows_per_w = h_hbm.shape[0] // nw
    base = wid * rows_per_w
    lane = lax.iota(jnp.int32, 16)
    lt8 = lane < T

    def do_row(r, acc):
        off = pl.multiple_of(r * 16, 16)
        kidx = idx_v[pl.ds(off, 16)]
        gval = gt_v[pl.ds(off, 16)]
        rvec = jnp.broadcast_to(r, (16,)).astype(jnp.int32)
        hval = plsc.load_gather(h_v, [rvec, kidx])
        tdv = plsc.load_gather(td_v, [rvec])
        vhv = plsc.load_gather(vh2_v, [rvec])
        mv = plsc.load_gather(mk_v, [rvec])
        gm = jnp.where(lt8, gval, 0.0)
        s = jnp.broadcast_to(jnp.sum(gm), (16,))
        gn = gm / (s + 1e-10)
        one_m_gn = 1.0 - gn
        mt = M * one_m_gn * one_m_gn
        y = jnp.maximum(vhv + hval, 0.0) + 1e-8
        i0 = np.int32(0x5F3759DF) - lax.shift_right_arithmetic(
            plsc.bitcast(y, jnp.int32), 1)
        rs = plsc.bitcast(i0, jnp.float32)
        for _ in range(3):
            rs = rs * (1.5 - 0.5 * y * rs * rs)
        dist = y * rs
        term = jnp.maximum(mt + tdv - dist, 0.0) * (1.0 / T)
        return acc + jnp.where(lt8, term, 0.0) * mv

    acc = jnp.zeros((16,), jnp.float32)
    nchunk = rows_per_w // SC_CHUNK
    for ci in range(nchunk):
        rowbase = base + ci * SC_CHUNK
        pltpu.sync_copy(h_hbm.at[pl.ds(rowbase, SC_CHUNK)], h_v)
        pltpu.sync_copy(gt_hbm.at[pl.ds(rowbase * 16, SC_CHUNK * 16)], gt_v)
        pltpu.sync_copy(idx_hbm.at[pl.ds(rowbase * 16, SC_CHUNK * 16)], idx_v)
        pltpu.sync_copy(td_hbm.at[pl.ds(rowbase, SC_CHUNK)], td_v)
        pltpu.sync_copy(vh2_hbm.at[pl.ds(rowbase, SC_CHUNK)], vh2_v)
        pltpu.sync_copy(mk_hbm.at[pl.ds(rowbase, SC_CHUNK)], mk_v)
        acc = lax.fori_loop(0, SC_CHUNK, do_row, acc)
    out_v[...] = acc
    pltpu.sync_copy(out_v, part_hbm.at[pl.ds(wid * 16, 16)])


def _tc_b(part_ref, ortho_ref, csum_ref, msum_ref, out_ref):
    jts = jnp.sum(part_ref[...])
    val = (csum_ref[0, 0] + jts) / jnp.maximum(msum_ref[0, 0], 1.0) \
        + LAM * ortho_ref[0, 0]
    out_ref[...] = jnp.broadcast_to(val, (1, 1))


@functools.partial(jax.jit, static_argnames=())
def kernel(v, vhat, d, g, F, negatives, mask):
    del d
    B, D = v.shape
    K = F.shape[0]
    N = negatives.shape[0]
    nblk = B // BLK
    maskf = mask.astype(jnp.float32)

    mesh = plsc.VectorSubcoreMesh(core_axis_name="c", subcore_axis_name="s")
    nw = 32

    gt_flat, idx_flat = pl.kernel(
        _sc_topk,
        mesh=mesh,
        compiler_params=pltpu.CompilerParams(needs_layout_passes=False),
        out_type=[
            jax.ShapeDtypeStruct((B * 16,), jnp.float32),
            jax.ShapeDtypeStruct((B * 16,), jnp.int32),
        ],
        scratch_types=[
            pltpu.VMEM((SC_CHUNK, K), jnp.float32),
            pltpu.VMEM((SC_CHUNK * 16,), jnp.float32),
            pltpu.VMEM((SC_CHUNK * 16,), jnp.int32),
        ],
    )(g)

    h, td, vh2, ortho, csum, msum = pl.pallas_call(
        _tc_a,
        grid=(nblk,),
        in_specs=[
            pl.BlockSpec((BLK, D), lambda i: (i, 0)),
            pl.BlockSpec((BLK, D), lambda i: (i, 0)),
            pl.BlockSpec((K, D), lambda i: (0, 0)),
            pl.BlockSpec((N, D), lambda i: (0, 0)),
            pl.BlockSpec((1, 1, BLK), lambda i: (i, 0, 0)),
        ],
        out_specs=[
            pl.BlockSpec((BLK, K), lambda i: (i, 0)),
            pl.BlockSpec((BLK,), lambda i: (i,)),
            pl.BlockSpec((BLK,), lambda i: (i,)),
            pl.BlockSpec((1, 1), lambda i: (0, 0)),
            pl.BlockSpec((1, 1), lambda i: (0, 0)),
            pl.BlockSpec((1, 1), lambda i: (0, 0)),
        ],
        out_shape=[
            jax.ShapeDtypeStruct((B, K), jnp.float32),
            jax.ShapeDtypeStruct((B,), jnp.float32),
            jax.ShapeDtypeStruct((B,), jnp.float32),
            jax.ShapeDtypeStruct((1, 1), jnp.float32),
            jax.ShapeDtypeStruct((1, 1), jnp.float32),
            jax.ShapeDtypeStruct((1, 1), jnp.float32),
        ],
        scratch_shapes=[pltpu.SMEM((3,), jnp.float32)],
    )(v, vhat, F, negatives, maskf.reshape(nblk, 1, BLK))

    partials = pl.kernel(
        _sc_jt,
        mesh=mesh,
        compiler_params=pltpu.CompilerParams(needs_layout_passes=False),
        out_type=jax.ShapeDtypeStruct((nw * 16,), jnp.float32),
        scratch_types=[
            pltpu.VMEM((SC_CHUNK, K), jnp.float32),
            pltpu.VMEM((SC_CHUNK * 16,), jnp.float32),
            pltpu.VMEM((SC_CHUNK * 16,), jnp.int32),
            pltpu.VMEM((SC_CHUNK,), jnp.float32),
            pltpu.VMEM((SC_CHUNK,), jnp.float32),
            pltpu.VMEM((SC_CHUNK,), jnp.float32),
            pltpu.VMEM((16,), jnp.float32),
        ],
    )(h, gt_flat, idx_flat, td, vh2, maskf)

    out = pl.pallas_call(
        _tc_b,
        in_specs=[
            pl.BlockSpec((nw * 16,), lambda: (0,)),
            pl.BlockSpec((1, 1), lambda: (0, 0)),
            pl.BlockSpec((1, 1), lambda: (0, 0)),
            pl.BlockSpec((1, 1), lambda: (0, 0)),
        ],
        out_specs=pl.BlockSpec((1, 1), lambda: (0, 0)),
        out_shape=jax.ShapeDtypeStruct((1, 1), jnp.float32),
    )(partials, ortho, csum, msum)
    return out.reshape(())


# speedup vs baseline: 1.6053x; 1.3017x over previous
import functools

import jax
import jax.numpy as jnp
import numpy as np
from jax import lax
from jax.experimental import pallas as pl
from jax.experimental.pallas import tpu as pltpu
from jax.experimental.pallas import tpu_sc as plsc

T = 8
M = 1.0
LAM = 0.01
BLK = 512
SC_CHUNK = 64


def _tc_a(v_ref, vh_ref, f_ref, neg_ref, mask_ref,
          h_ref, stats_ref, ortho_ref, csum_ref, msum_ref, acc):
    pid = pl.program_id(0)
    nblk = pl.num_programs(0)

    @pl.when(pid == 0)
    def _init():
        f = f_ref[...]
        gram = lax.dot_general(f, f, (((1,), (1,)), ((), ())),
                               preferred_element_type=jnp.float32)
        k = gram.shape[0]
        rows = lax.broadcasted_iota(jnp.int32, gram.shape, 0)
        cols = lax.broadcasted_iota(jnp.int32, gram.shape, 1)
        tr = jnp.sum(jnp.where(rows == cols, gram, 0.0))
        acc[0] = jnp.sum(gram * gram) - 2.0 * tr + float(k)
        acc[1] = 0.0
        acc[2] = 0.0

    vhat = vh_ref[...]
    v = v_ref[...]
    vh2 = jnp.sum(vhat * vhat, axis=1)
    td = jnp.sqrt(jnp.sum((vhat - v) ** 2, axis=1) + 1e-8)

    ones_row = jnp.ones((8, v.shape[1]), jnp.float32)
    neg = neg_ref[...]
    nn2 = lax.dot_general(ones_row, neg * neg, (((1,), (1,)), ((), ())),
                          preferred_element_type=jnp.float32)[0:1, :]
    ndots = lax.dot_general(vhat, neg, (((1,), (1,)), ((), ())),
                            preferred_element_type=jnp.float32)
    nd = jnp.sqrt(jnp.maximum(vh2[:, None] - 2.0 * ndots + nn2, 0.0) + 1e-8)
    c = jnp.mean(jnp.maximum(1.0 + td[:, None] - nd, 0.0), axis=1)

    f = f_ref[...]
    fn2 = lax.dot_general(ones_row, f * f, (((1,), (1,)), ((), ())),
                          preferred_element_type=jnp.float32)[0:1, :]
    dots = lax.dot_general(vhat, f, (((1,), (1,)), ((), ())),
                           preferred_element_type=jnp.float32)
    h_ref[...] = fn2 - 2.0 * dots

    zcol = jnp.zeros_like(td)
    stats_ref[...] = jnp.stack(
        [td, vh2, zcol, zcol, zcol, zcol, zcol, zcol], axis=1)

    mask = mask_ref[0, 0, :]
    acc[1] += jnp.sum(mask * c)
    acc[2] += jnp.sum(mask)

    @pl.when(pid == nblk - 1)
    def _fin():
        ortho_ref[...] = jnp.broadcast_to(acc[0], (1, 1))
        csum_ref[...] = jnp.broadcast_to(acc[1], (1, 1))
        msum_ref[...] = jnp.broadcast_to(acc[2], (1, 1))


def _sc_topk(g_hbm, gt_hbm, idx_hbm, g_v0, g_v1, gt_v, idx_v, sem0, sem1):
    g_bufs = [g_v0, g_v1]
    sem_bufs = [sem0, sem1]
    info = plsc.get_sparse_core_info()
    nc = info.num_cores
    nw = nc * info.num_subcores
    wid = lax.axis_index("s") * nc + lax.axis_index("c")
    rows_per_w = g_hbm.shape[0] // nw
    base = wid * rows_per_w
    kk = g_hbm.shape[1]
    nvec = kk // 16
    lane = lax.iota(jnp.int32, 16)

    def topk_row(gref, r):
        sorted_vecs = []
        for j in range(nvec):
            gv = gref[r, pl.ds(j * 16, 16)]
            kv = (plsc.bitcast(gv, jnp.int32) & np.int32(~511)) \
                | (lane + np.int32(j * 16))
            sorted_vecs.append(jnp.sort(kv))
        while len(sorted_vecs) > 1:
            nxt = []
            for a, b in zip(sorted_vecs[0::2], sorted_vecs[1::2]):
                low = jnp.minimum(a, jnp.flip(b, 0))
                nxt.append(jnp.sort(low))
            sorted_vecs = nxt
        return sorted_vecs[0]

    nchunk = rows_per_w // SC_CHUNK

    def issue(ci):
        buf = ci % 2
        rowbase = base + ci * SC_CHUNK
        return pltpu.async_copy(
            g_hbm.at[pl.ds(rowbase, SC_CHUNK)], g_bufs[buf], sem_bufs[buf])

    cp = issue(0)
    for ci in range(nchunk):
        buf = ci % 2
        nxt = issue(ci + 1) if ci + 1 < nchunk else None
        cp.wait()
        cp = nxt
        rowbase = base + ci * SC_CHUNK

        def do_row(r, carry, _buf=buf):
            best = topk_row(g_bufs[_buf], r)
            gval = plsc.bitcast(best & np.int32(~511), jnp.float32)
            off = pl.multiple_of(r * 16, 16)
            gt_v[pl.ds(off, 16)] = gval
            idx_v[pl.ds(off, 16)] = best & np.int32(511)
            return carry

        lax.fori_loop(0, SC_CHUNK, do_row, 0)
        pltpu.sync_copy(gt_v, gt_hbm.at[pl.ds(rowbase * 16, SC_CHUNK * 16)])
        pltpu.sync_copy(idx_v, idx_hbm.at[pl.ds(rowbase * 16, SC_CHUNK * 16)])


def _sc_jt(h_hbm, gt_hbm, idx_hbm, stats_hbm, mk_hbm, part_hbm,
           h_v0, h_v1, gt_v0, gt_v1, idx_v0, idx_v1, st_v0, st_v1,
           mk_v0, mk_v1, out_v, sem0, sem1):
    h_bufs = [h_v0, h_v1]
    gt_bufs = [gt_v0, gt_v1]
    idx_bufs = [idx_v0, idx_v1]
    st_bufs = [st_v0, st_v1]
    mk_bufs = [mk_v0, mk_v1]
    sem_bufs = [sem0, sem1]
    info = plsc.get_sparse_core_info()
    nc = info.num_cores
    nw = nc * info.num_subcores
    wid = lax.axis_index("s") * nc + lax.axis_index("c")
    rows_per_w = h_hbm.shape[0] // nw
    base = wid * rows_per_w
    lane = lax.iota(jnp.int32, 16)
    lt8 = lane < T
    zero16 = jnp.zeros((16,), jnp.int32)
    one16 = jnp.broadcast_to(jnp.int32(1), (16,))

    def do_row(href, gtref, idxref, stref, mkref, r, acc):
        off = pl.multiple_of(r * 16, 16)
        kidx = idxref[pl.ds(off, 16)]
        gval = gtref[pl.ds(off, 16)]
        rvec = jnp.broadcast_to(r, (16,)).astype(jnp.int32)
        hval = plsc.load_gather(href, [rvec, kidx])
        tdv = plsc.load_gather(stref, [rvec, zero16])
        vhv = plsc.load_gather(stref, [rvec, one16])
        mv = plsc.load_gather(mkref, [rvec])
        gm = jnp.where(lt8, gval, 0.0)
        s = jnp.broadcast_to(jnp.sum(gm), (16,))
        gn = gm / (s + 1e-10)
        one_m_gn = 1.0 - gn
        mt = M * one_m_gn * one_m_gn
        y = jnp.maximum(vhv + hval, 0.0) + 1e-8
        i0 = np.int32(0x5F3759DF) - lax.shift_right_arithmetic(
            plsc.bitcast(y, jnp.int32), 1)
        rs = plsc.bitcast(i0, jnp.float32)
        for _ in range(3):
            rs = rs * (1.5 - 0.5 * y * rs * rs)
        dist = y * rs
        term = jnp.maximum(mt + tdv - dist, 0.0) * (1.0 / T)
        return acc + jnp.where(lt8, term, 0.0) * mv

    nchunk = rows_per_w // SC_CHUNK

    def issue(ci):
        buf = ci % 2
        rowbase = base + ci * SC_CHUNK
        sem = sem_bufs[buf]
        return [
            pltpu.async_copy(h_hbm.at[pl.ds(rowbase, SC_CHUNK)],
                             h_bufs[buf], sem),
            pltpu.async_copy(gt_hbm.at[pl.ds(rowbase * 16, SC_CHUNK * 16)],
                             gt_bufs[buf], sem),
            pltpu.async_copy(idx_hbm.at[pl.ds(rowbase * 16, SC_CHUNK * 16)],
                             idx_bufs[buf], sem),
            pltpu.async_copy(stats_hbm.at[pl.ds(rowbase, SC_CHUNK)],
                             st_bufs[buf], sem),
            pltpu.async_copy(mk_hbm.at[pl.ds(rowbase, SC_CHUNK)],
                             mk_bufs[buf], sem),
        ]

    acc = jnp.zeros((16,), jnp.float32)
    cps = issue(0)
    for ci in range(nchunk):
        buf = ci % 2
        nxt = issue(ci + 1) if ci + 1 < nchunk else None
        for cp in cps:
            cp.wait()
        cps = nxt

        def row2(r, a, _buf=buf):
            a = do_row(h_bufs[_buf], gt_bufs[_buf], idx_bufs[_buf],
                       st_bufs[_buf], mk_bufs[_buf], 2 * r, a)
            return do_row(h_bufs[_buf], gt_bufs[_buf], idx_bufs[_buf],
                          st_bufs[_buf], mk_bufs[_buf], 2 * r + 1, a)

        acc = lax.fori_loop(0, SC_CHUNK // 2, row2, acc)
    out_v[...] = acc
    pltpu.sync_copy(out_v, part_hbm.at[pl.ds(wid * 16, 16)])


def _tc_b(part_ref, ortho_ref, csum_ref, msum_ref, out_ref):
    jts = jnp.sum(part_ref[...])
    val = (csum_ref[0, 0] + jts) / jnp.maximum(msum_ref[0, 0], 1.0) \
        + LAM * ortho_ref[0, 0]
    out_ref[...] = jnp.broadcast_to(val, (1, 1))


@functools.partial(jax.jit, static_argnames=())
def kernel(v, vhat, d, g, F, negatives, mask):
    del d
    B, D = v.shape
    K = F.shape[0]
    N = negatives.shape[0]
    nblk = B // BLK
    maskf = mask.astype(jnp.float32)

    mesh = plsc.VectorSubcoreMesh(core_axis_name="c", subcore_axis_name="s")
    nw = 32

    gt_flat, idx_flat = pl.kernel(
        _sc_topk,
        mesh=mesh,
        compiler_params=pltpu.CompilerParams(needs_layout_passes=False),
        out_type=[
            jax.ShapeDtypeStruct((B * 16,), jnp.float32),
            jax.ShapeDtypeStruct((B * 16,), jnp.int32),
        ],
        scratch_types=[
            pltpu.VMEM((SC_CHUNK, K), jnp.float32),
            pltpu.VMEM((SC_CHUNK, K), jnp.float32),
            pltpu.VMEM((SC_CHUNK * 16,), jnp.float32),
            pltpu.VMEM((SC_CHUNK * 16,), jnp.int32),
            pltpu.SemaphoreType.DMA,
            pltpu.SemaphoreType.DMA,
        ],
    )(g)

    h, stats, ortho, csum, msum = pl.pallas_call(
        _tc_a,
        grid=(nblk,),
        in_specs=[
            pl.BlockSpec((BLK, D), lambda i: (i, 0)),
            pl.BlockSpec((BLK, D), lambda i: (i, 0)),
            pl.BlockSpec((K, D), lambda i: (0, 0)),
            pl.BlockSpec((N, D), lambda i: (0, 0)),
            pl.BlockSpec((1, 1, BLK), lambda i: (i, 0, 0)),
        ],
        out_specs=[
            pl.BlockSpec((BLK, K), lambda i: (i, 0)),
            pl.BlockSpec((BLK, 8), lambda i: (i, 0)),
            pl.BlockSpec((1, 1), lambda i: (0, 0)),
            pl.BlockSpec((1, 1), lambda i: (0, 0)),
            pl.BlockSpec((1, 1), lambda i: (0, 0)),
        ],
        out_shape=[
            jax.ShapeDtypeStruct((B, K), jnp.float32),
            jax.ShapeDtypeStruct((B, 8), jnp.float32),
            jax.ShapeDtypeStruct((1, 1), jnp.float32),
            jax.ShapeDtypeStruct((1, 1), jnp.float32),
            jax.ShapeDtypeStruct((1, 1), jnp.float32),
        ],
        scratch_shapes=[pltpu.SMEM((3,), jnp.float32)],
    )(v, vhat, F, negatives, maskf.reshape(nblk, 1, BLK))

    partials = pl.kernel(
        _sc_jt,
        mesh=mesh,
        compiler_params=pltpu.CompilerParams(needs_layout_passes=False),
        out_type=jax.ShapeDtypeStruct((nw * 16,), jnp.float32),
        scratch_types=[
            pltpu.VMEM((SC_CHUNK, K), jnp.float32),
            pltpu.VMEM((SC_CHUNK, K), jnp.float32),
            pltpu.VMEM((SC_CHUNK * 16,), jnp.float32),
            pltpu.VMEM((SC_CHUNK * 16,), jnp.float32),
            pltpu.VMEM((SC_CHUNK * 16,), jnp.int32),
            pltpu.VMEM((SC_CHUNK * 16,), jnp.int32),
            pltpu.VMEM((SC_CHUNK, 8), jnp.float32),
            pltpu.VMEM((SC_CHUNK, 8), jnp.float32),
            pltpu.VMEM((SC_CHUNK,), jnp.float32),
            pltpu.VMEM((SC_CHUNK,), jnp.float32),
            pltpu.VMEM((16,), jnp.float32),
            pltpu.SemaphoreType.DMA,
            pltpu.SemaphoreType.DMA,
        ],
    )(h, gt_flat, idx_flat, stats, maskf)

    out = pl.pallas_call(
        _tc_b,
        in_specs=[
            pl.BlockSpec((nw * 16,), lambda: (0,)),
            pl.BlockSpec((1, 1), lambda: (0, 0)),
            pl.BlockSpec((1, 1), lambda: (0, 0)),
            pl.BlockSpec((1, 1), lambda: (0, 0)),
        ],
        out_specs=pl.BlockSpec((1, 1), lambda: (0, 0)),
        out_shape=jax.ShapeDtypeStruct((1, 1), jnp.float32),
    )(partials, ortho, csum, msum)
    return out.reshape(())
